# baseline (speedup 1.0000x reference)
import functools

import jax
import jax.numpy as jnp
import numpy as np
from jax import lax
from jax.experimental import layout as jax_layout
from jax.experimental import pallas as pl
from jax.experimental.pallas import tpu as pltpu
from jax.experimental.pallas import tpu_sc as plsc

B, L, V, D = 4096, 200, 100000, 128
LP = 208
LANES = 16
NC, NS = 2, 16
NW = NC * NS
BPW = B // NW
G0, G1 = 112, LP - 112
UNROLL = 16
NCH = D // LANES
DW = D // 2
ZPAD = 32768


def _sc_sums(embw, idxs):
    mesh = plsc.VectorSubcoreMesh(core_axis_name="c", subcore_axis_name="s")

    @functools.partial(
        pl.kernel,
        out_type=jax.ShapeDtypeStruct((B, D), jnp.float32),
        mesh=mesh,
        scratch_types=[
            pltpu.VMEM((BPW * LP,), jnp.int32),
            pltpu.VMEM((LP, DW), jnp.int32),
            pltpu.VMEM((LP, DW), jnp.int32),
            pltpu.VMEM((BPW, D), jnp.float32),
            pltpu.SemaphoreType.DMA,
            pltpu.SemaphoreType.DMA,
        ],
    )
    def k(emb_hbm, idx_hbm, out_hbm, idx_v, rows0, rows1, out_v, sem0, sem1):
        wid = lax.axis_index("s") * NC + lax.axis_index("c")
        base = wid * BPW
        pltpu.sync_copy(idx_hbm.at[pl.ds(base * LP, BPW * LP)], idx_v)
        himask = jnp.full((LANES,), -65536, jnp.int32)

        def issue(b, rows, sem):
            pltpu.async_copy(emb_hbm.at[idx_v.at[pl.ds(b * LP, G0)]],
                             rows.at[pl.ds(0, G0)], sem)
            pltpu.async_copy(emb_hbm.at[idx_v.at[pl.ds(b * LP + G0, G1)]],
                             rows.at[pl.ds(G0, G1)], sem)

        def drain(b, rows, sem):
            pltpu.make_async_copy(emb_hbm.at[idx_v.at[pl.ds(b * LP, G0)]],
                                  rows.at[pl.ds(0, G0)], sem).wait()
            pltpu.make_async_copy(emb_hbm.at[idx_v.at[pl.ds(b * LP + G0, G1)]],
                                  rows.at[pl.ds(G0, G1)], sem).wait()

        def accum(b, rows):
            def rowstep(j, acc):
                r0 = j * UNROLL
                acc = list(acc)
                for u in range(UNROLL):
                    for g in range(NCH // 2):
                        xi = rows[r0 + u, pl.ds(LANES * g, LANES)]
                        acc[2 * g] = acc[2 * g] + lax.bitcast_convert_type(
                            xi << 16, jnp.float32)
                        acc[2 * g + 1] = (acc[2 * g + 1]
                                          + lax.bitcast_convert_type(
                                              xi & himask, jnp.float32))
                return tuple(acc)

            acc = lax.fori_loop(
                0, LP // UNROLL, rowstep,
                tuple(jnp.zeros((LANES,), jnp.float32) for _ in range(NCH)))
            for c in range(NCH):
                out_v[b, pl.ds(c * LANES, LANES)] = acc[c]

        issue(0, rows0, sem0)

        def pair(g, carry):
            b0 = 2 * g
            b1 = 2 * g + 1
            issue(b1, rows1, sem1)
            drain(b0, rows0, sem0)
            accum(b0, rows0)
            issue(lax.rem(b0 + 2, BPW), rows0, sem0)
            drain(b1, rows1, sem1)
            accum(b1, rows1)
            return carry

        lax.fori_loop(0, BPW // 2, pair, 0)
        drain(0, rows0, sem0)
        pltpu.sync_copy(out_v, out_hbm.at[pl.ds(base, BPW)])

    return k(embw, idxs)


def _tc_finish(sums, mask, proj_tp):
    BB = 512

    def body(s_ref, m_ref, p_ref, o_ref):
        cnt = jnp.sum(m_ref[...].astype(jnp.float32), axis=1, keepdims=True)
        y = jnp.dot(s_ref[...], p_ref[...], preferred_element_type=jnp.float32)
        o_ref[...] = y / jnp.clip(cnt, 1e-9, None)

    return pl.pallas_call(
        body,
        grid=(B // BB,),
        in_specs=[
            pl.BlockSpec((BB, D), lambda i: (i, 0)),
            pl.BlockSpec((BB, L), lambda i: (i, 0)),
            pl.BlockSpec((D, D), lambda i: (0, 0)),
        ],
        out_specs=pl.BlockSpec((BB, D), lambda i: (i, 0)),
        out_shape=jax.ShapeDtypeStruct((B, D), jnp.float32),
    )(sums, mask, proj_tp)


_QIDX = np.array([32 * (c // 2) + 2 * l + (c % 2)
                  for c in range(NCH) for l in range(LANES)])


def kernel(input_ids, attention_mask, emb, proj):
    ids = input_ids.astype(jnp.int32)
    msk = attention_mask.astype(jnp.int32)
    embb = jnp.concatenate(
        [emb.astype(jnp.bfloat16),
         jnp.zeros((ZPAD, D), jnp.bfloat16)], axis=0)
    embw = lax.bitcast_convert_type(
        embb.reshape(V + ZPAD, DW, 2), jnp.int32)
    embw = jax_layout.with_layout_constraint(
        embw, jax_layout.Layout(major_to_minor=(0, 1), tiling=((16,),)))
    mskp = jnp.pad(msk, ((0, 0), (0, LP - L)))
    spread = V + (jnp.arange(B * LP, dtype=jnp.int32) % ZPAD).reshape(B, LP)
    idxs = jnp.where(mskp == 1, jnp.pad(ids, ((0, 0), (0, LP - L))), spread)
    sums = _sc_sums(embw, idxs.reshape(B * LP))
    return _tc_finish(sums, msk, proj.T[_QIDX, :])
